# baseline (speedup 1.0000x reference)
import functools

import jax
import jax.numpy as jnp
import numpy as np
from jax import lax
from jax.experimental import pallas as pl
from jax.experimental.pallas import tpu as pltpu
from jax.experimental.pallas import tpu_sc as plsc

_SAMPLE_TO = 1000
_LANES = 16
_N_CHUNKS = 10


@functools.lru_cache(maxsize=None)
def _plan(k: int):
  with jax.ensure_compile_time_eval():
    ridxs = np.asarray(jax.random.permutation(jax.random.key(42), k))
  ridxs = ridxs[:_SAMPLE_TO].astype(np.int64)
  need = int(ridxs.max()) + 1
  w = 128 * (-(-need // (128 * _N_CHUNKS)))
  assert _N_CHUNKS * w <= k, "uniform chunking must stay in bounds"
  cols, pos = [], []
  for c in range(_N_CHUNKS):
    sel = np.where((ridxs >= c * w) & (ridxs < (c + 1) * w))[0]
    cols.append(ridxs[sel] - c * w)
    pos.append(sel)
  p = max(len(x) for x in cols)
  p = -(-p // _LANES) * _LANES
  cols_arr = np.zeros((_N_CHUNKS, p), np.int32)
  pos_arr = np.zeros((_N_CHUNKS, p), np.int32)
  for c in range(_N_CHUNKS):
    cols_arr[c] = cols[c][0]
    pos_arr[c] = pos[c][0]
    cols_arr[c, : len(cols[c])] = cols[c]
    pos_arr[c, : len(pos[c])] = pos[c]
  return cols_arr, pos_arr, p, w


def _build_kernel(rows: int, k: int, p: int, w: int):
  info = plsc.get_sparse_core_info()
  nw = info.num_cores * info.num_subcores
  n_groups = rows // 8
  gpw = n_groups // nw

  mesh = plsc.VectorSubcoreMesh(core_axis_name="c", subcore_axis_name="s")

  @functools.partial(
      pl.kernel,
      mesh=mesh,
      out_type=jax.ShapeDtypeStruct((rows, _SAMPLE_TO), jnp.float32),
      compiler_params=pltpu.CompilerParams(needs_layout_passes=False),
      scratch_types=[
          pltpu.VMEM((8, w), jnp.float32),
          pltpu.VMEM((8, _SAMPLE_TO), jnp.float32),
          pltpu.VMEM((_N_CHUNKS, p), jnp.int32),
          pltpu.VMEM((_N_CHUNKS, p), jnp.int32),
      ],
  )
  def ds_kernel(in_hbm, cols_hbm, pos_hbm, out_hbm, chunk_v, out_v,
                cols_v, pos_v):
    wid = lax.axis_index("s") * info.num_cores + lax.axis_index("c")
    pltpu.sync_copy(cols_hbm, cols_v)
    pltpu.sync_copy(pos_hbm, pos_v)

    def per_group(gi, _):
      g8 = pl.multiple_of((wid * gpw + gi) * 8, 8)

      def per_chunk(c, _):
        off = pl.multiple_of(c * w, 128)
        pltpu.sync_copy(in_hbm.at[pl.ds(g8, 8), pl.ds(off, w)], chunk_v)

        def per_vec(t, _):
          cv = cols_v[c, pl.ds(t * _LANES, _LANES)]
          pv = pos_v[c, pl.ds(t * _LANES, _LANES)]
          for r in range(8):
            rv = jnp.full((_LANES,), r, jnp.int32)
            vals = plsc.load_gather(chunk_v, [rv, cv])
            plsc.store_scatter(out_v, [rv, pv], vals)
          return _

        return lax.fori_loop(0, p // _LANES, per_vec, _)

      lax.fori_loop(0, _N_CHUNKS, per_chunk, None)
      pltpu.sync_copy(out_v, out_hbm.at[pl.ds(g8, 8)])
      return _

    lax.fori_loop(0, gpw, per_group, None)

  return ds_kernel


def kernel(inputs):
  rows, k = inputs.shape
  if k <= _SAMPLE_TO:
    return inputs
  cols_arr, pos_arr, p, w = _plan(k)
  fn = _build_kernel(rows, k, p, w)
  return fn(inputs, jnp.asarray(cols_arr), jnp.asarray(pos_arr))


# speedup vs baseline: 19.3417x; 19.3417x over previous
import functools

import jax
import jax.numpy as jnp
import numpy as np
from jax import lax
from jax.experimental import pallas as pl
from jax.experimental.pallas import tpu as pltpu
from jax.experimental.pallas import tpu_sc as plsc

_SAMPLE_TO = 1000


@functools.lru_cache(maxsize=None)
def _plan(k: int):
  with jax.ensure_compile_time_eval():
    ridxs = np.asarray(jax.random.permutation(jax.random.key(42), k))
  return ridxs[:_SAMPLE_TO].astype(np.int32).reshape(_SAMPLE_TO // 8, 8)


def _build_kernel(k: int, d: int):
  info = plsc.get_sparse_core_info()
  nw = info.num_cores * info.num_subcores
  ng = _SAMPLE_TO // 8
  gpw = -(-ng // nw)

  mesh = plsc.VectorSubcoreMesh(core_axis_name="c", subcore_axis_name="s")

  @functools.partial(
      pl.kernel,
      mesh=mesh,
      out_type=jax.ShapeDtypeStruct((_SAMPLE_TO, d), jnp.float32),
      compiler_params=pltpu.CompilerParams(needs_layout_passes=False),
      scratch_types=[
          pltpu.VMEM((ng, 8), jnp.int32),
          pltpu.VMEM((8, d), jnp.float32),
          pltpu.SemaphoreType.DMA,
      ],
  )
  def gather_kernel(t_hbm, idx_hbm, out_hbm, idx_v, rows_v, sem):
    wid = lax.axis_index("s") * info.num_cores + lax.axis_index("c")
    pltpu.sync_copy(idx_hbm, idx_v)

    def per_tile(gi, _):
      g = wid + nw * gi

      @pl.when(g < ng)
      def _do():
        pltpu.async_copy(t_hbm.at[idx_v.at[g]], rows_v, sem).wait()
        pltpu.sync_copy(rows_v, out_hbm.at[pl.ds(pl.multiple_of(g * 8, 8), 8)])

      return _

    lax.fori_loop(0, gpw, per_tile, None)

  return gather_kernel


def kernel(inputs):
  rows, k = inputs.shape
  if k <= _SAMPLE_TO:
    return inputs
  idx = _plan(k)
  fn = _build_kernel(k, rows)
  out_t = fn(inputs.T, jnp.asarray(idx))
  return out_t.T


# speedup vs baseline: 21.1079x; 1.0913x over previous
import functools

import jax
import jax.numpy as jnp
import numpy as np
from jax import lax
from jax.experimental import pallas as pl
from jax.experimental.pallas import tpu as pltpu
from jax.experimental.pallas import tpu_sc as plsc

_SAMPLE_TO = 1000


@functools.lru_cache(maxsize=None)
def _plan(k: int):
  with jax.ensure_compile_time_eval():
    ridxs = np.asarray(jax.random.permutation(jax.random.key(42), k))
  return ridxs[:_SAMPLE_TO].astype(np.int32).reshape(_SAMPLE_TO // 8, 8)


def _build_kernel(k: int, d: int):
  info = plsc.get_sparse_core_info()
  nw = info.num_cores * info.num_subcores
  ng = _SAMPLE_TO // 8
  gpw = -(-ng // nw)

  mesh = plsc.VectorSubcoreMesh(core_axis_name="c", subcore_axis_name="s")

  @functools.partial(
      pl.kernel,
      mesh=mesh,
      out_type=jax.ShapeDtypeStruct((_SAMPLE_TO, d), jnp.float32),
      compiler_params=pltpu.CompilerParams(needs_layout_passes=False),
      scratch_types=[
          pltpu.VMEM((ng, 8), jnp.int32),
          pltpu.VMEM((gpw, 8, d), jnp.float32),
          pltpu.SemaphoreType.DMA,
      ],
  )
  def gather_kernel(t_hbm, idx_hbm, out_hbm, idx_v, rows_v, sem):
    wid = lax.axis_index("s") * info.num_cores + lax.axis_index("c")
    pltpu.sync_copy(idx_hbm, idx_v)

    for gi in range(gpw):
      g = wid + nw * gi

      @pl.when(g < ng)
      def _fire():
        pltpu.async_copy(t_hbm.at[idx_v.at[g]], rows_v.at[gi], sem)

    for gi in range(gpw):
      g = wid + nw * gi

      @pl.when(g < ng)
      def _drain():
        pltpu.make_async_copy(t_hbm.at[idx_v.at[g]], rows_v.at[gi],
                              sem).wait()
        pltpu.sync_copy(rows_v.at[gi],
                        out_hbm.at[pl.ds(pl.multiple_of(g * 8, 8), 8)])

  return gather_kernel


def kernel(inputs):
  rows, k = inputs.shape
  if k <= _SAMPLE_TO:
    return inputs
  idx = _plan(k)
  fn = _build_kernel(k, rows)
  out_t = fn(inputs.T, jnp.asarray(idx))
  return out_t.T
